# baseline (speedup 1.0000x reference)
import jax
import jax.numpy as jnp
from jax.experimental import pallas as pl
from jax.experimental.pallas import tpu as pltpu

N = 10000
D = 512
TILE_M = 400
NI = N // TILE_M
KB16 = 1936
KF = N - KB16
TILE_T = 1000


def _xform_kernel(h_ref, w_ref, b_ref, t_ref):
    acc = jnp.dot(h_ref[...], w_ref[...], preferred_element_type=jnp.float32)
    t_ref[...] = (acc + b_ref[...]).astype(jnp.bfloat16)


def _l1_kernel(hg_ref, t_ref, w_ref, b_ref, tn_ref, hgc_ref):
    hg16 = hg_ref[...].astype(jnp.bfloat16)
    hgc_ref[...] = hg16[:, KF:]
    acc = jnp.dot(hg16, t_ref[...], preferred_element_type=jnp.float32)
    h = jnp.maximum(acc, 0.0).astype(jnp.bfloat16)
    tn = jnp.dot(h, w_ref[...], preferred_element_type=jnp.float32)
    tn_ref[...] = (tn + b_ref[...]).astype(jnp.bfloat16)


def _split_dot(hgf_ref, hgc_ref, t_ref):
    acc = jnp.dot(hgf_ref[...].astype(jnp.bfloat16), t_ref[:KF, :],
                  preferred_element_type=jnp.float32)
    return acc + jnp.dot(hgc_ref[...], t_ref[KF:, :],
                         preferred_element_type=jnp.float32)


def _mid_kernel(hgf_ref, hgc_ref, t_ref, w_ref, b_ref, tn_ref):
    acc = _split_dot(hgf_ref, hgc_ref, t_ref)
    h = jnp.maximum(acc, 0.0).astype(jnp.bfloat16)
    tn = jnp.dot(h, w_ref[...], preferred_element_type=jnp.float32)
    tn_ref[...] = (tn + b_ref[...]).astype(jnp.bfloat16)


def _last_kernel(hgf_ref, hgc_ref, t_ref, out_ref):
    acc = _split_dot(hgf_ref, hgc_ref, t_ref)
    out_ref[...] = jnp.maximum(acc, 0.0)


def _consumer_specs():
    return [
        pl.BlockSpec((TILE_M, KF), lambda i: (i, 0)),
        pl.BlockSpec((TILE_M, KB16), lambda i: (i, 0)),
        pl.BlockSpec((N, D), lambda i: (0, 0)),
    ]


def kernel(x, hg, W1, b1, W2, b2, W3, b3):
    t1 = pl.pallas_call(
        _xform_kernel,
        grid=(N // TILE_T,),
        in_specs=[
            pl.BlockSpec((TILE_T, D), lambda i: (i, 0)),
            pl.BlockSpec((D, D), lambda i: (0, 0)),
            pl.BlockSpec((1, D), lambda i: (0, 0)),
        ],
        out_specs=pl.BlockSpec((TILE_T, D), lambda i: (i, 0)),
        out_shape=jax.ShapeDtypeStruct((N, D), jnp.bfloat16),
        compiler_params=pltpu.CompilerParams(
            dimension_semantics=("parallel",)),
    )(x, W1, b1.reshape(1, D))

    t2, hgc = pl.pallas_call(
        _l1_kernel,
        grid=(NI,),
        in_specs=[
            pl.BlockSpec((TILE_M, N), lambda i: (i, 0)),
            pl.BlockSpec((N, D), lambda i: (0, 0)),
            pl.BlockSpec((D, D), lambda i: (0, 0)),
            pl.BlockSpec((1, D), lambda i: (0, 0)),
        ],
        out_specs=[
            pl.BlockSpec((TILE_M, D), lambda i: (i, 0)),
            pl.BlockSpec((TILE_M, KB16), lambda i: (i, 0)),
        ],
        out_shape=[
            jax.ShapeDtypeStruct((N, D), jnp.bfloat16),
            jax.ShapeDtypeStruct((N, KB16), jnp.bfloat16),
        ],
        compiler_params=pltpu.CompilerParams(
            dimension_semantics=("arbitrary",)),
    )(hg, t1, W2.astype(jnp.bfloat16), b2.reshape(1, D))

    t3 = pl.pallas_call(
        _mid_kernel,
        grid=(NI,),
        in_specs=_consumer_specs() + [
            pl.BlockSpec((D, D), lambda i: (0, 0)),
            pl.BlockSpec((1, D), lambda i: (0, 0)),
        ],
        out_specs=pl.BlockSpec((TILE_M, D), lambda i: (i, 0)),
        out_shape=jax.ShapeDtypeStruct((N, D), jnp.bfloat16),
        compiler_params=pltpu.CompilerParams(
            dimension_semantics=("arbitrary",)),
    )(hg, hgc, t2, W3.astype(jnp.bfloat16), b3.reshape(1, D))

    return pl.pallas_call(
        _last_kernel,
        grid=(NI,),
        in_specs=_consumer_specs(),
        out_specs=pl.BlockSpec((TILE_M, D), lambda i: (i, 0)),
        out_shape=jax.ShapeDtypeStruct((N, D), jnp.float32),
        compiler_params=pltpu.CompilerParams(
            dimension_semantics=("arbitrary",)),
    )(hg, hgc, t3)


# speedup vs baseline: 1.0084x; 1.0084x over previous
import jax
import jax.numpy as jnp
from jax.experimental import pallas as pl
from jax.experimental.pallas import tpu as pltpu

N = 10000
D = 512
TILE_M = 400
NI = N // TILE_M
KB16 = 2448
KF = N - KB16
TILE_T = 2000


def _xform_kernel(h_ref, w_ref, b_ref, t_ref):
    acc = jnp.dot(h_ref[...], w_ref[...], preferred_element_type=jnp.float32)
    t_ref[...] = (acc + b_ref[...]).astype(jnp.bfloat16)


def _l1_kernel(hg_ref, t_ref, w_ref, b_ref, tn_ref, hgc_ref):
    hg16 = hg_ref[...].astype(jnp.bfloat16)
    hgc_ref[...] = hg16[:, KF:]
    acc = jnp.dot(hg16, t_ref[...], preferred_element_type=jnp.float32)
    h = jnp.maximum(acc, 0.0).astype(jnp.bfloat16)
    tn = jnp.dot(h, w_ref[...], preferred_element_type=jnp.float32)
    tn_ref[...] = (tn + b_ref[...]).astype(jnp.bfloat16)


def _split_dot(hgf_ref, hgc_ref, t_ref):
    acc = jnp.dot(hgf_ref[...].astype(jnp.bfloat16), t_ref[:KF, :],
                  preferred_element_type=jnp.float32)
    return acc + jnp.dot(hgc_ref[...], t_ref[KF:, :],
                         preferred_element_type=jnp.float32)


def _mid_kernel(hgf_ref, hgc_ref, t_ref, w_ref, b_ref, tn_ref):
    acc = _split_dot(hgf_ref, hgc_ref, t_ref)
    h = jnp.maximum(acc, 0.0).astype(jnp.bfloat16)
    tn = jnp.dot(h, w_ref[...], preferred_element_type=jnp.float32)
    tn_ref[...] = (tn + b_ref[...]).astype(jnp.bfloat16)


def _last_kernel(hgf_ref, hgc_ref, t_ref, out_ref):
    acc = _split_dot(hgf_ref, hgc_ref, t_ref)
    out_ref[...] = jnp.maximum(acc, 0.0)


def _consumer_specs():
    return [
        pl.BlockSpec((TILE_M, KF), lambda i: (i, 0)),
        pl.BlockSpec((TILE_M, KB16), lambda i: (i, 0)),
        pl.BlockSpec((N, D), lambda i: (0, 0)),
    ]


def kernel(x, hg, W1, b1, W2, b2, W3, b3):
    t1 = pl.pallas_call(
        _xform_kernel,
        grid=(N // TILE_T,),
        in_specs=[
            pl.BlockSpec((TILE_T, D), lambda i: (i, 0)),
            pl.BlockSpec((D, D), lambda i: (0, 0)),
            pl.BlockSpec((1, D), lambda i: (0, 0)),
        ],
        out_specs=pl.BlockSpec((TILE_T, D), lambda i: (i, 0)),
        out_shape=jax.ShapeDtypeStruct((N, D), jnp.bfloat16),
        compiler_params=pltpu.CompilerParams(
            dimension_semantics=("parallel",)),
    )(x, W1, b1.reshape(1, D))

    t2, hgc = pl.pallas_call(
        _l1_kernel,
        grid=(NI,),
        in_specs=[
            pl.BlockSpec((TILE_M, N), lambda i: (i, 0)),
            pl.BlockSpec((N, D), lambda i: (0, 0)),
            pl.BlockSpec((D, D), lambda i: (0, 0)),
            pl.BlockSpec((1, D), lambda i: (0, 0)),
        ],
        out_specs=[
            pl.BlockSpec((TILE_M, D), lambda i: (i, 0)),
            pl.BlockSpec((TILE_M, KB16), lambda i: (i, 0)),
        ],
        out_shape=[
            jax.ShapeDtypeStruct((N, D), jnp.bfloat16),
            jax.ShapeDtypeStruct((N, KB16), jnp.bfloat16),
        ],
        compiler_params=pltpu.CompilerParams(
            dimension_semantics=("arbitrary",)),
    )(hg, t1, W2.astype(jnp.bfloat16), b2.reshape(1, D))

    t3 = pl.pallas_call(
        _mid_kernel,
        grid=(NI,),
        in_specs=_consumer_specs() + [
            pl.BlockSpec((D, D), lambda i: (0, 0)),
            pl.BlockSpec((1, D), lambda i: (0, 0)),
        ],
        out_specs=pl.BlockSpec((TILE_M, D), lambda i: (i, 0)),
        out_shape=jax.ShapeDtypeStruct((N, D), jnp.bfloat16),
        compiler_params=pltpu.CompilerParams(
            dimension_semantics=("arbitrary",)),
    )(hg, hgc, t2, W3.astype(jnp.bfloat16), b3.reshape(1, D))

    return pl.pallas_call(
        _last_kernel,
        grid=(NI,),
        in_specs=_consumer_specs(),
        out_specs=pl.BlockSpec((TILE_M, D), lambda i: (i, 0)),
        out_shape=jax.ShapeDtypeStruct((N, D), jnp.float32),
        compiler_params=pltpu.CompilerParams(
            dimension_semantics=("arbitrary",)),
    )(hg, hgc, t3)
